# baseline (speedup 1.0000x reference)
import jax
import jax.numpy as jnp
from jax.experimental import pallas as pl
from jax.experimental.pallas import tpu as pltpu

_N = 4096
_D = 128
_BR = 512
_GB = 4096
_NBLK = _N // _GB
_NSUB = _N // _BR
_TEMP = 1.0


def _loss_kernel(zi_ref, zj_ref, out_ref, pos_s, nneg_s, rsum_s):
    i = pl.program_id(0)
    zj = zj_ref[...]
    csum = jnp.sum(zj.astype(jnp.float32), axis=0, keepdims=True)

    for h in range(_GB // _BR):
        zi = zi_ref[pl.ds(h * _BR, _BR), :]
        simT = jax.lax.dot_general(
            zj, zi, (((1,), (1,)), ((), ())),
            preferred_element_type=jnp.float32)
        pos = jnp.max(simT, axis=0, keepdims=True)
        mask = (simT < pos).astype(jnp.bfloat16)
        row_sum = jax.lax.dot_general(
            csum, zi.astype(jnp.float32), (((1,), (1,)), ((), ())),
            preferred_element_type=jnp.float32)
        num_neg = jax.lax.dot_general(
            jnp.ones((1, _N), jnp.bfloat16), mask, (((1,), (0,)), ((), ())),
            preferred_element_type=jnp.float32)

        blk = pl.ds(i * (_GB // _BR) + h, 1)
        pos_s[blk, :] = pos / _TEMP
        nneg_s[blk, :] = num_neg
        rsum_s[blk, :] = row_sum / _TEMP

    @pl.when(i == _NBLK - 1)
    def _finalize():
        pos_a = pos_s[...]
        nneg_a = nneg_s[...]
        rsum_a = rsum_s[...]
        neg_sum = rsum_a - pos_a * (_N - nneg_a)
        max_neg = jnp.max(nneg_a)
        neg_mean = neg_sum / max_neg
        exp_pos = jnp.exp(pos_a)
        exp_neg = jnp.exp(jnp.minimum(neg_sum, 30.0))
        info_nce = -jnp.mean(jnp.log(exp_pos / exp_neg))
        triplet = jnp.mean(jnp.maximum(pos_a - neg_mean + 1.0, 0.0))
        out_ref[...] = jnp.reshape(info_nce + triplet, (1, 1))


def kernel(z_i, z_j):
    out = pl.pallas_call(
        _loss_kernel,
        grid=(_NBLK,),
        in_specs=[
            pl.BlockSpec((_GB, _D), lambda i: (i, 0)),
            pl.BlockSpec((_N, _D), lambda i: (0, 0)),
        ],
        out_specs=pl.BlockSpec((1, 1), lambda i: (0, 0)),
        out_shape=jax.ShapeDtypeStruct((1, 1), jnp.float32),
        scratch_shapes=[
            pltpu.VMEM((_NSUB, _BR), jnp.float32),
            pltpu.VMEM((_NSUB, _BR), jnp.float32),
            pltpu.VMEM((_NSUB, _BR), jnp.float32),
        ],
    )(z_i.astype(jnp.bfloat16), z_j.astype(jnp.bfloat16))
    return out[0, 0]


# speedup vs baseline: 1.2555x; 1.2555x over previous
import jax
import jax.numpy as jnp
from jax.experimental import pallas as pl
from jax.experimental.pallas import tpu as pltpu

_N = 4096
_D = 128
_BR = 512
_GB = 4096
_NBLK = _N // _GB
_NSUB = _N // _BR
_TEMP = 1.0


def _loss_kernel(zi_ref, zj_ref, out_ref, pos_s, nneg_s, rsum_s):
    i = pl.program_id(0)
    zj = zj_ref[...]
    csum = jnp.sum(zj, axis=0, keepdims=True)

    for h in range(_GB // _BR):
        zi = zi_ref[pl.ds(h * _BR, _BR), :]
        simT = jax.lax.dot_general(
            zj, zi, (((1,), (1,)), ((), ())),
            preferred_element_type=jnp.float32)
        pos = jnp.max(simT, axis=0, keepdims=True)
        row_sum = jax.lax.dot_general(
            csum, zi, (((1,), (1,)), ((), ())),
            preferred_element_type=jnp.float32)
        num_neg = jnp.sum((simT < pos).astype(jnp.float32), axis=0,
                          keepdims=True)

        blk = pl.ds(i * (_GB // _BR) + h, 1)
        pos_s[blk, :] = pos / _TEMP
        nneg_s[blk, :] = num_neg
        rsum_s[blk, :] = row_sum / _TEMP

    @pl.when(i == _NBLK - 1)
    def _finalize():
        pos_a = pos_s[...]
        nneg_a = nneg_s[...]
        rsum_a = rsum_s[...]
        neg_sum = rsum_a - pos_a * (_N - nneg_a)
        max_neg = jnp.max(nneg_a)
        neg_mean = neg_sum / max_neg
        exp_pos = jnp.exp(pos_a)
        exp_neg = jnp.exp(jnp.minimum(neg_sum, 30.0))
        info_nce = -jnp.mean(jnp.log(exp_pos / exp_neg))
        triplet = jnp.mean(jnp.maximum(pos_a - neg_mean + 1.0, 0.0))
        out_ref[...] = jnp.reshape(info_nce + triplet, (1, 1))


def kernel(z_i, z_j):
    out = pl.pallas_call(
        _loss_kernel,
        grid=(_NBLK,),
        in_specs=[
            pl.BlockSpec((_GB, _D), lambda i: (i, 0)),
            pl.BlockSpec((_N, _D), lambda i: (0, 0)),
        ],
        out_specs=pl.BlockSpec((1, 1), lambda i: (0, 0)),
        out_shape=jax.ShapeDtypeStruct((1, 1), jnp.float32),
        scratch_shapes=[
            pltpu.VMEM((_NSUB, _BR), jnp.float32),
            pltpu.VMEM((_NSUB, _BR), jnp.float32),
            pltpu.VMEM((_NSUB, _BR), jnp.float32),
        ],
    )(z_i, z_j)
    return out[0, 0]
